# Initial kernel scaffold; baseline (speedup 1.0000x reference)
import jax
import jax.numpy as jnp
from jax.experimental import pallas as pl


def kernel(x, edge_index, W, b):
    raise NotImplementedError("write your pallas kernel here")


# speedup vs baseline: 29.9743x; 29.9743x over previous
import functools

import jax
import jax.numpy as jnp
import numpy as np
from jax import lax
from jax.experimental import pallas as pl
from jax.experimental.pallas import tpu as pltpu
from jax.experimental.pallas import tpu_sc as plsc

N = 10000
E = 320000
D = 128
NP = 10016
NC = 2
NS = 16
NW = NC * NS
K = 128
CCH = 79
EPW = K * CCH
EPAD = NW * EPW
NPA = 10240
RPW = NPA // NS

_mesh = plsc.VectorSubcoreMesh(core_axis_name="c", subcore_axis_name="s")


@functools.partial(
    pl.kernel,
    out_type=jax.ShapeDtypeStruct((NC * NP,), jnp.float32),
    mesh=_mesh,
    scratch_types=[
        pltpu.VMEM((CCH, K), jnp.int32),
        pltpu.VMEM((K,), jnp.float32),
        pltpu.VMEM((NP,), jnp.float32),
        pltpu.VMEM_SHARED((NP,), jnp.float32),
    ],
)
def _deg_pass(dst_hbm, out_hbm, dst_v, ones_v, zero_v, acc_sh):
    c = lax.axis_index("c")
    s = lax.axis_index("s")
    wid = c * NS + s
    pltpu.sync_copy(dst_hbm.at[wid], dst_v)
    for i in range(K // 16):
        ones_v[pl.ds(16 * i, 16)] = jnp.ones((16,), jnp.float32)

    @pl.when(s == 0)
    def _zero():
        def zbody(i, carry):
            zero_v[pl.ds(i * 16, 16)] = jnp.zeros((16,), jnp.float32)
            return carry

        lax.fori_loop(0, NP // 16, zbody, 0)
        pltpu.sync_copy(zero_v, acc_sh)

    plsc.subcore_barrier()

    def body(j, carry):
        pltpu.sync_copy(ones_v, acc_sh.at[dst_v.at[j]], add=True)
        return carry

    lax.fori_loop(0, CCH, body, 0)
    plsc.subcore_barrier()

    @pl.when(s == 0)
    def _writeout():
        pltpu.sync_copy(acc_sh, zero_v)
        pltpu.sync_copy(zero_v, out_hbm.at[pl.ds(c * NP, NP)])


@functools.partial(
    pl.kernel,
    out_type=jax.ShapeDtypeStruct((NC * NPA, D), jnp.float32),
    mesh=_mesh,
    scratch_types=[
        pltpu.VMEM((CCH, K), jnp.int32),
        pltpu.VMEM((CCH, K), jnp.int32),
        pltpu.VMEM((K, D), jnp.float32),
        pltpu.VMEM_SHARED((NPA, D), jnp.float32),
        pltpu.SemaphoreType.DMA,
    ],
)
def _agg_pass(g_hbm, src_hbm, dst_hbm, out_hbm, src_v, dst_v, rows_v,
              acc_sh, sem):
    c = lax.axis_index("c")
    s = lax.axis_index("s")
    wid = c * NS + s
    pltpu.sync_copy(src_hbm.at[wid], src_v)
    pltpu.sync_copy(dst_hbm.at[wid], dst_v)

    def zbody(i, carry):
        for jj in range(D // 16):
            rows_v[i, pl.ds(jj * 16, 16)] = jnp.zeros((16,), jnp.float32)
        return carry

    lax.fori_loop(0, K, zbody, 0)
    for t in range(RPW // K):
        pltpu.sync_copy(rows_v, acc_sh.at[pl.ds(s * RPW + t * K, K)])
    plsc.subcore_barrier()

    def body(j, carry):
        pltpu.async_copy(g_hbm.at[src_v.at[j]], rows_v, sem).wait()
        pltpu.sync_copy(rows_v, acc_sh.at[dst_v.at[j]], add=True)
        return carry

    lax.fori_loop(0, CCH, body, 0)
    plsc.subcore_barrier()
    pltpu.sync_copy(acc_sh.at[pl.ds(s * RPW, RPW)],
                    out_hbm.at[pl.ds(c * NPA + s * RPW, RPW)])


def _dense1_body(x_ref, w_ref, d0_ref, d1_ref, g_ref):
    deg = d0_ref[0:N, :] + d1_ref[0:N, :] + 1.0
    dis = lax.rsqrt(deg)
    h = lax.dot_general(x_ref[...], w_ref[...], (((1,), (1,)), ((), ())),
                        precision=lax.Precision.HIGHEST,
                        preferred_element_type=jnp.float32)
    g_ref[0:N, :] = dis * h
    g_ref[N:NP, :] = jnp.zeros((NP - N, D), jnp.float32)


_dense1 = pl.pallas_call(
    _dense1_body,
    out_shape=jax.ShapeDtypeStruct((NP, D), jnp.float32),
)


def _dense2_body(a0_ref, a1_ref, g_ref, d0_ref, d1_ref, b_ref, o_ref):
    deg = d0_ref[0:N, :] + d1_ref[0:N, :] + 1.0
    dis = lax.rsqrt(deg)
    tot = a0_ref[...] + a1_ref[...] + g_ref[0:N, :]
    o_ref[...] = jnp.maximum(dis * tot + b_ref[...], 0.0)


_dense2 = pl.pallas_call(
    _dense2_body,
    out_shape=jax.ShapeDtypeStruct((N, D), jnp.float32),
)


_PAD_IDX = np.arange(EPAD - E, dtype=np.int32) % 16 + N


def kernel(x, edge_index, W, b):
    src = edge_index[0].astype(jnp.int32)
    dst = edge_index[1].astype(jnp.int32)
    padi = jnp.asarray(_PAD_IDX)
    src3 = jnp.concatenate([src, padi]).reshape(NW, CCH, K)
    dst3 = jnp.concatenate([dst, padi]).reshape(NW, CCH, K)

    degf = _deg_pass(dst3)
    d0 = degf[:NP].reshape(NP, 1)
    d1 = degf[NP:].reshape(NP, 1)

    g = _dense1(x, W, d0, d1)
    accf = _agg_pass(g, src3, dst3)
    out = _dense2(accf[:N], accf[NPA:NPA + N], g, d0, d1, b.reshape(1, D))
    return out


# speedup vs baseline: 34.3187x; 1.1449x over previous
import functools

import jax
import jax.numpy as jnp
import numpy as np
from jax import lax
from jax.experimental import pallas as pl
from jax.experimental.pallas import tpu as pltpu
from jax.experimental.pallas import tpu_sc as plsc

N = 10000
E = 320000
D = 128
NP = 10016
NC = 2
NS = 16
NW = NC * NS
K = 128
BCH = 8
NB = 10
CCH = BCH * NB
EPW = K * CCH
EPAD = NW * EPW
NPA = 10240
RPW = NPA // NS

_mesh = plsc.VectorSubcoreMesh(core_axis_name="c", subcore_axis_name="s")


@functools.partial(
    pl.kernel,
    out_type=jax.ShapeDtypeStruct((NC * NP,), jnp.float32),
    mesh=_mesh,
    scratch_types=[
        pltpu.VMEM((CCH, K), jnp.int32),
        pltpu.VMEM((K,), jnp.float32),
        pltpu.VMEM((NP,), jnp.float32),
        pltpu.VMEM_SHARED((NP,), jnp.float32),
        pltpu.SemaphoreType.DMA,
    ],
)
def _deg_pass(dst_hbm, out_hbm, dst_v, ones_v, zero_v, acc_sh, dsem):
    c = lax.axis_index("c")
    s = lax.axis_index("s")
    wid = c * NS + s
    pltpu.sync_copy(dst_hbm.at[wid], dst_v)
    for i in range(K // 16):
        ones_v[pl.ds(16 * i, 16)] = jnp.ones((16,), jnp.float32)

    @pl.when(s == 0)
    def _zero():
        def zbody(i, carry):
            zero_v[pl.ds(i * 16, 16)] = jnp.zeros((16,), jnp.float32)
            return carry

        lax.fori_loop(0, NP // 16, zbody, 0)
        pltpu.sync_copy(zero_v, acc_sh)

    plsc.subcore_barrier()

    def fire(j, carry):
        pltpu.async_copy(ones_v, acc_sh.at[dst_v.at[j]], dsem, add=True)
        return carry

    lax.fori_loop(0, CCH, fire, 0)

    def drain(j, carry):
        pltpu.make_async_copy(ones_v, acc_sh.at[dst_v.at[j]], dsem).wait()
        return carry

    lax.fori_loop(0, CCH, drain, 0)
    plsc.subcore_barrier()

    @pl.when(s == 0)
    def _writeout():
        pltpu.sync_copy(acc_sh, zero_v)
        pltpu.sync_copy(zero_v, out_hbm.at[pl.ds(c * NP, NP)])


@functools.partial(
    pl.kernel,
    out_type=jax.ShapeDtypeStruct((NC * NPA, D), jnp.float32),
    mesh=_mesh,
    scratch_types=[
        pltpu.VMEM((BCH, K), jnp.int32),
        pltpu.VMEM((BCH, K), jnp.int32),
        pltpu.VMEM((K, D), jnp.float32),
        pltpu.VMEM((K, D), jnp.float32),
        pltpu.VMEM_SHARED((NPA, D), jnp.float32),
        pltpu.SemaphoreType.DMA,
        pltpu.SemaphoreType.DMA,
        pltpu.SemaphoreType.DMA,
        pltpu.SemaphoreType.DMA,
    ],
)
def _agg_pass(g_hbm, src_hbm, dst_hbm, out_hbm, srcb, dstb, rows0, rows1,
              acc_sh, gs0, gs1, ss0, ss1):
    c = lax.axis_index("c")
    s = lax.axis_index("s")
    wid = c * NS + s
    rows = (rows0, rows1)
    gs = (gs0, gs1)
    ss = (ss0, ss1)

    def zbody(i, carry):
        for jj in range(D // 16):
            rows0[i, pl.ds(jj * 16, 16)] = jnp.zeros((16,), jnp.float32)
        return carry

    lax.fori_loop(0, K, zbody, 0)
    for t in range(RPW // K):
        pltpu.sync_copy(rows0, acc_sh.at[pl.ds(s * RPW + t * K, K)])
    plsc.subcore_barrier()

    def block(i, carry):
        base = wid * NB + i
        pltpu.sync_copy(src_hbm.at[base], srcb)
        pltpu.sync_copy(dst_hbm.at[base], dstb)
        pltpu.async_copy(g_hbm.at[srcb.at[0]], rows0, gs0)
        for b in range(1, BCH):
            pb = (b - 1) % 2
            cb = b % 2
            pltpu.make_async_copy(g_hbm.at[srcb.at[b - 1]], rows[pb],
                                  gs[pb]).wait()
            pltpu.async_copy(rows[pb], acc_sh.at[dstb.at[b - 1]], ss[pb],
                             add=True)
            if b >= 2:
                pltpu.make_async_copy(rows[cb], acc_sh.at[dstb.at[b - 2]],
                                      ss[cb]).wait()
            pltpu.async_copy(g_hbm.at[srcb.at[b]], rows[cb], gs[cb])
        lb = (BCH - 1) % 2
        pltpu.make_async_copy(g_hbm.at[srcb.at[BCH - 1]], rows[lb],
                              gs[lb]).wait()
        pltpu.async_copy(rows[lb], acc_sh.at[dstb.at[BCH - 1]], ss[lb],
                         add=True)
        pltpu.make_async_copy(rows[1 - lb], acc_sh.at[dstb.at[BCH - 2]],
                              ss[1 - lb]).wait()
        pltpu.make_async_copy(rows[lb], acc_sh.at[dstb.at[BCH - 1]],
                              ss[lb]).wait()
        return carry

    lax.fori_loop(0, NB, block, 0)
    plsc.subcore_barrier()
    pltpu.sync_copy(acc_sh.at[pl.ds(s * RPW, RPW)],
                    out_hbm.at[pl.ds(c * NPA + s * RPW, RPW)])


def _dense1_body(x_ref, w_ref, d0_ref, d1_ref, g_ref):
    deg = d0_ref[0:N, :] + d1_ref[0:N, :] + 1.0
    dis = lax.rsqrt(deg)
    h = lax.dot_general(x_ref[...], w_ref[...], (((1,), (1,)), ((), ())),
                        precision=lax.Precision.HIGHEST,
                        preferred_element_type=jnp.float32)
    g_ref[0:N, :] = dis * h
    g_ref[N:NP, :] = jnp.zeros((NP - N, D), jnp.float32)


_dense1 = pl.pallas_call(
    _dense1_body,
    out_shape=jax.ShapeDtypeStruct((NP, D), jnp.float32),
)


def _dense2_body(a0_ref, a1_ref, g_ref, d0_ref, d1_ref, b_ref, o_ref):
    deg = d0_ref[0:N, :] + d1_ref[0:N, :] + 1.0
    dis = lax.rsqrt(deg)
    tot = a0_ref[...] + a1_ref[...] + g_ref[0:N, :]
    o_ref[...] = jnp.maximum(dis * tot + b_ref[...], 0.0)


_dense2 = pl.pallas_call(
    _dense2_body,
    out_shape=jax.ShapeDtypeStruct((N, D), jnp.float32),
)


_PAD_IDX = np.arange(EPAD - E, dtype=np.int32) % 16 + N


def kernel(x, edge_index, W, b):
    src = edge_index[0].astype(jnp.int32)
    dst = edge_index[1].astype(jnp.int32)
    padi = jnp.asarray(_PAD_IDX)
    srcp = jnp.concatenate([src, padi])
    dstp = jnp.concatenate([dst, padi])

    degf = _deg_pass(dstp.reshape(NW, CCH, K))
    d0 = degf[:NP].reshape(NP, 1)
    d1 = degf[NP:].reshape(NP, 1)

    g = _dense1(x, W, d0, d1)
    accf = _agg_pass(g, srcp.reshape(NW * NB, BCH, K),
                     dstp.reshape(NW * NB, BCH, K))
    out = _dense2(accf[:N], accf[NPA:NPA + N], g, d0, d1, b.reshape(1, D))
    return out


# speedup vs baseline: 37.5718x; 1.0948x over previous
import functools

import jax
import jax.numpy as jnp
import numpy as np
from jax import lax
from jax.experimental import pallas as pl
from jax.experimental.pallas import tpu as pltpu
from jax.experimental.pallas import tpu_sc as plsc

N = 10000
E = 320000
D = 128
NP = 10016
NC = 2
NS = 16
NW = NC * NS
K = 128
CCH = 80
KA = 64
BCH = 16
NB = 10
NBUF = 4
EPW = KA * BCH * NB
EPAD = NW * EPW
NPA = 10240
RPW = NPA // NS

_mesh = plsc.VectorSubcoreMesh(core_axis_name="c", subcore_axis_name="s")


@functools.partial(
    pl.kernel,
    out_type=jax.ShapeDtypeStruct((NC * NP,), jnp.float32),
    mesh=_mesh,
    scratch_types=[
        pltpu.VMEM((CCH, K), jnp.int32),
        pltpu.VMEM((K,), jnp.float32),
        pltpu.VMEM((NP,), jnp.float32),
        pltpu.VMEM_SHARED((NP,), jnp.float32),
        pltpu.SemaphoreType.DMA,
    ],
)
def _deg_pass(dst_hbm, out_hbm, dst_v, ones_v, zero_v, acc_sh, dsem):
    c = lax.axis_index("c")
    s = lax.axis_index("s")
    wid = c * NS + s
    pltpu.sync_copy(dst_hbm.at[wid], dst_v)
    for i in range(K // 16):
        ones_v[pl.ds(16 * i, 16)] = jnp.ones((16,), jnp.float32)

    @pl.when(s == 0)
    def _zero():
        def zbody(i, carry):
            zero_v[pl.ds(i * 16, 16)] = jnp.zeros((16,), jnp.float32)
            return carry

        lax.fori_loop(0, NP // 16, zbody, 0)
        pltpu.sync_copy(zero_v, acc_sh)

    plsc.subcore_barrier()

    def fire(j, carry):
        pltpu.async_copy(ones_v, acc_sh.at[dst_v.at[j]], dsem, add=True)
        return carry

    lax.fori_loop(0, CCH, fire, 0)

    def drain(j, carry):
        pltpu.make_async_copy(ones_v, acc_sh.at[dst_v.at[j]], dsem).wait()
        return carry

    lax.fori_loop(0, CCH, drain, 0)
    plsc.subcore_barrier()

    @pl.when(s == 0)
    def _writeout():
        pltpu.sync_copy(acc_sh, zero_v)
        pltpu.sync_copy(zero_v, out_hbm.at[pl.ds(c * NP, NP)])


@functools.partial(
    pl.kernel,
    out_type=jax.ShapeDtypeStruct((NC * NPA, D), jnp.float32),
    mesh=_mesh,
    scratch_types=[
        pltpu.VMEM((BCH, KA), jnp.int32),
        pltpu.VMEM((BCH, KA), jnp.int32),
        pltpu.VMEM((KA, D), jnp.float32),
        pltpu.VMEM((KA, D), jnp.float32),
        pltpu.VMEM((KA, D), jnp.float32),
        pltpu.VMEM((KA, D), jnp.float32),
        pltpu.VMEM_SHARED((NPA, D), jnp.float32),
        pltpu.SemaphoreType.DMA,
        pltpu.SemaphoreType.DMA,
        pltpu.SemaphoreType.DMA,
        pltpu.SemaphoreType.DMA,
        pltpu.SemaphoreType.DMA,
        pltpu.SemaphoreType.DMA,
        pltpu.SemaphoreType.DMA,
        pltpu.SemaphoreType.DMA,
    ],
)
def _agg_pass(g_hbm, src_hbm, dst_hbm, out_hbm, srcb, dstb, rows0, rows1,
              rows2, rows3, acc_sh, gs0, gs1, gs2, gs3, ss0, ss1, ss2, ss3):
    c = lax.axis_index("c")
    s = lax.axis_index("s")
    wid = c * NS + s
    rows = (rows0, rows1, rows2, rows3)
    gs = (gs0, gs1, gs2, gs3)
    ss = (ss0, ss1, ss2, ss3)

    def zbody(i, carry):
        for jj in range(D // 16):
            rows0[i, pl.ds(jj * 16, 16)] = jnp.zeros((16,), jnp.float32)
            rows1[i, pl.ds(jj * 16, 16)] = jnp.zeros((16,), jnp.float32)
        return carry

    lax.fori_loop(0, KA, zbody, 0)
    for t in range(RPW // KA // 2):
        pltpu.sync_copy(rows0, acc_sh.at[pl.ds(s * RPW + (2 * t) * KA, KA)])
        pltpu.sync_copy(rows1,
                        acc_sh.at[pl.ds(s * RPW + (2 * t + 1) * KA, KA)])
    plsc.subcore_barrier()

    def _g(b, buf):
        return pltpu.async_copy(g_hbm.at[srcb.at[b]], rows[buf], gs[buf])

    def _wg(b, buf):
        pltpu.make_async_copy(g_hbm.at[srcb.at[b]], rows[buf],
                              gs[buf]).wait()

    def _s(b, buf):
        return pltpu.async_copy(rows[buf], acc_sh.at[dstb.at[b]], ss[buf],
                                add=True)

    def _ws(b, buf):
        pltpu.make_async_copy(rows[buf], acc_sh.at[dstb.at[b]],
                              ss[buf]).wait()

    def block(i, carry):
        base = wid * NB + i
        pltpu.sync_copy(src_hbm.at[base], srcb)
        pltpu.sync_copy(dst_hbm.at[base], dstb)
        for p in range(NBUF - 1):
            _g(p, p)
        for b in range(BCH):
            _wg(b, b % NBUF)
            _s(b, b % NBUF)
            nb = b + NBUF - 1
            if nb < BCH:
                if b >= 1:
                    _ws(b - 1, (b - 1) % NBUF)
                _g(nb, nb % NBUF)
        for b in range(BCH - NBUF, BCH):
            _ws(b, b % NBUF)
        return carry

    lax.fori_loop(0, NB, block, 0)
    plsc.subcore_barrier()
    pltpu.sync_copy(acc_sh.at[pl.ds(s * RPW, RPW)],
                    out_hbm.at[pl.ds(c * NPA + s * RPW, RPW)])


def _dense0_body(x_ref, w_ref, h_ref):
    h_ref[...] = lax.dot_general(x_ref[...], w_ref[...],
                                 (((1,), (1,)), ((), ())),
                                 precision=lax.Precision.HIGHEST,
                                 preferred_element_type=jnp.float32)


_dense0 = pl.pallas_call(
    _dense0_body,
    out_shape=jax.ShapeDtypeStruct((N, D), jnp.float32),
)


def _dense1_body(h_ref, d0_ref, d1_ref, g_ref):
    deg = d0_ref[0:N, :] + d1_ref[0:N, :] + 1.0
    dis = lax.rsqrt(deg)
    g_ref[0:N, :] = dis * h_ref[...]
    g_ref[N:NP, :] = jnp.zeros((NP - N, D), jnp.float32)


_dense1 = pl.pallas_call(
    _dense1_body,
    out_shape=jax.ShapeDtypeStruct((NP, D), jnp.float32),
)


def _dense2_body(a0_ref, a1_ref, g_ref, d0_ref, d1_ref, b_ref, o_ref):
    deg = d0_ref[0:N, :] + d1_ref[0:N, :] + 1.0
    dis = lax.rsqrt(deg)
    tot = a0_ref[...] + a1_ref[...] + g_ref[0:N, :]
    o_ref[...] = jnp.maximum(dis * tot + b_ref[...], 0.0)


_dense2 = pl.pallas_call(
    _dense2_body,
    out_shape=jax.ShapeDtypeStruct((N, D), jnp.float32),
)


_PAD_IDX = np.arange(EPAD - E, dtype=np.int32) % 16 + N


def kernel(x, edge_index, W, b):
    src = edge_index[0].astype(jnp.int32)
    dst = edge_index[1].astype(jnp.int32)
    padi = jnp.asarray(_PAD_IDX)
    srcp = jnp.concatenate([src, padi])
    dstp = jnp.concatenate([dst, padi])

    h = _dense0(x, W)
    degf = _deg_pass(dstp.reshape(NW, CCH, K))
    d0 = degf[:NP].reshape(NP, 1)
    d1 = degf[NP:].reshape(NP, 1)

    g = _dense1(h, d0, d1)
    accf = _agg_pass(g, srcp.reshape(NW * NB, BCH, KA),
                     dstp.reshape(NW * NB, BCH, KA))
    out = _dense2(accf[:N], accf[NPA:NPA + N], g, d0, d1, b.reshape(1, D))
    return out


# speedup vs baseline: 42.0736x; 1.1198x over previous
import functools

import jax
import jax.numpy as jnp
import numpy as np
from jax import lax
from jax.experimental import pallas as pl
from jax.experimental.pallas import tpu as pltpu
from jax.experimental.pallas import tpu_sc as plsc

N = 10000
E = 320000
D = 128
NP = 10016
NC = 2
NS = 16
NW = NC * NS
K = 128
CCH = 80
KA = 64
BCH = 16
NB = 10
NBUF = 4
EPW = KA * BCH * NB
EPAD = NW * EPW
NPA = 10240
RPW = NPA // NS

_mesh = plsc.VectorSubcoreMesh(core_axis_name="c", subcore_axis_name="s")


@functools.partial(
    pl.kernel,
    out_type=jax.ShapeDtypeStruct((NC * NP,), jnp.float32),
    mesh=_mesh,
    scratch_types=[
        pltpu.VMEM((CCH, K), jnp.int32),
        pltpu.VMEM((K,), jnp.float32),
        pltpu.VMEM((NP,), jnp.float32),
        pltpu.VMEM_SHARED((NP,), jnp.float32),
        pltpu.SemaphoreType.DMA,
    ],
)
def _deg_pass(dst_hbm, out_hbm, dst_v, ones_v, zero_v, acc_sh, dsem):
    c = lax.axis_index("c")
    s = lax.axis_index("s")
    wid = c * NS + s
    pltpu.sync_copy(dst_hbm.at[pl.ds(wid * CCH, CCH)], dst_v)
    for i in range(K // 16):
        ones_v[pl.ds(16 * i, 16)] = jnp.ones((16,), jnp.float32)

    @pl.when(s == 0)
    def _zero():
        def zbody(i, carry):
            zero_v[pl.ds(i * 16, 16)] = jnp.zeros((16,), jnp.float32)
            return carry

        lax.fori_loop(0, NP // 16, zbody, 0)
        pltpu.sync_copy(zero_v, acc_sh)

    plsc.subcore_barrier()

    def fire(j, carry):
        pltpu.async_copy(ones_v, acc_sh.at[dst_v.at[j]], dsem, add=True)
        return carry

    lax.fori_loop(0, CCH, fire, 0)

    def drain(j, carry):
        pltpu.make_async_copy(ones_v, acc_sh.at[dst_v.at[j]], dsem).wait()
        return carry

    lax.fori_loop(0, CCH, drain, 0)
    plsc.subcore_barrier()

    @pl.when(s == 0)
    def _writeout():
        pltpu.sync_copy(acc_sh, zero_v)
        pltpu.sync_copy(zero_v, out_hbm.at[pl.ds(c * NP, NP)])


@functools.partial(
    pl.kernel,
    out_type=jax.ShapeDtypeStruct((NC * NPA, D), jnp.float32),
    mesh=_mesh,
    scratch_types=[
        pltpu.VMEM((8, K), jnp.int32),
        pltpu.VMEM((8, K), jnp.int32),
        pltpu.VMEM((BCH, KA), jnp.int32),
        pltpu.VMEM((BCH, KA), jnp.int32),
        pltpu.VMEM((KA, D), jnp.float32),
        pltpu.VMEM((KA, D), jnp.float32),
        pltpu.VMEM((KA, D), jnp.float32),
        pltpu.VMEM((KA, D), jnp.float32),
        pltpu.VMEM_SHARED((NPA, D), jnp.float32),
        pltpu.SemaphoreType.DMA,
        pltpu.SemaphoreType.DMA,
        pltpu.SemaphoreType.DMA,
        pltpu.SemaphoreType.DMA,
        pltpu.SemaphoreType.DMA,
        pltpu.SemaphoreType.DMA,
        pltpu.SemaphoreType.DMA,
        pltpu.SemaphoreType.DMA,
    ],
)
def _agg_pass(g_hbm, src_hbm, dst_hbm, out_hbm, srcr, dstr, srcb, dstb,
              rows0, rows1, rows2, rows3, acc_sh, gs0, gs1, gs2, gs3,
              ss0, ss1, ss2, ss3):
    c = lax.axis_index("c")
    s = lax.axis_index("s")
    wid = c * NS + s
    rows = (rows0, rows1, rows2, rows3)
    gs = (gs0, gs1, gs2, gs3)
    ss = (ss0, ss1, ss2, ss3)

    def zbody(i, carry):
        for jj in range(D // 16):
            rows0[i, pl.ds(jj * 16, 16)] = jnp.zeros((16,), jnp.float32)
            rows1[i, pl.ds(jj * 16, 16)] = jnp.zeros((16,), jnp.float32)
        return carry

    lax.fori_loop(0, KA, zbody, 0)
    for t in range(RPW // KA // 2):
        pltpu.sync_copy(rows0, acc_sh.at[pl.ds(s * RPW + (2 * t) * KA, KA)])
        pltpu.sync_copy(rows1,
                        acc_sh.at[pl.ds(s * RPW + (2 * t + 1) * KA, KA)])
    plsc.subcore_barrier()

    def _g(b, buf):
        return pltpu.async_copy(g_hbm.at[srcb.at[b]], rows[buf], gs[buf])

    def _wg(b, buf):
        pltpu.make_async_copy(g_hbm.at[srcb.at[b]], rows[buf],
                              gs[buf]).wait()

    def _s(b, buf):
        return pltpu.async_copy(rows[buf], acc_sh.at[dstb.at[b]], ss[buf],
                                add=True)

    def _ws(b, buf):
        pltpu.make_async_copy(rows[buf], acc_sh.at[dstb.at[b]],
                              ss[buf]).wait()

    def block(i, carry):
        base = (wid * NB + i) * 8
        pltpu.sync_copy(src_hbm.at[pl.ds(base, 8)], srcr)
        pltpu.sync_copy(dst_hbm.at[pl.ds(base, 8)], dstr)
        for ch in range(BCH):
            r, hh = ch // 2, ch % 2
            for q in range(KA // 16):
                srcb[ch, pl.ds(16 * q, 16)] = srcr[r,
                                                   pl.ds(64 * hh + 16 * q, 16)]
                dstb[ch, pl.ds(16 * q, 16)] = dstr[r,
                                                   pl.ds(64 * hh + 16 * q, 16)]
        for p in range(NBUF - 1):
            _g(p, p)
        for b in range(BCH):
            _wg(b, b % NBUF)
            _s(b, b % NBUF)
            nb = b + NBUF - 1
            if nb < BCH:
                if b >= 1:
                    _ws(b - 1, (b - 1) % NBUF)
                _g(nb, nb % NBUF)
        for b in range(BCH - NBUF, BCH):
            _ws(b, b % NBUF)
        return carry

    lax.fori_loop(0, NB, block, 0)
    plsc.subcore_barrier()
    pltpu.sync_copy(acc_sh.at[pl.ds(s * RPW, RPW)],
                    out_hbm.at[pl.ds(c * NPA + s * RPW, RPW)])


def _dense0_body(x_ref, w_ref, h_ref):
    h_ref[...] = lax.dot_general(x_ref[...], w_ref[...],
                                 (((1,), (1,)), ((), ())),
                                 precision=lax.Precision.HIGHEST,
                                 preferred_element_type=jnp.float32)


_dense0 = pl.pallas_call(
    _dense0_body,
    out_shape=jax.ShapeDtypeStruct((N, D), jnp.float32),
)


def _dis_col(d_ref):
    deg = d_ref[0:1, 0:N] + d_ref[1:2, 0:N] + 1.0
    return lax.transpose(lax.rsqrt(deg), (1, 0))


def _dense1_body(h_ref, d_ref, g_ref):
    g_ref[0:N, :] = _dis_col(d_ref) * h_ref[...]
    g_ref[N:NP, :] = jnp.zeros((NP - N, D), jnp.float32)


_dense1 = pl.pallas_call(
    _dense1_body,
    out_shape=jax.ShapeDtypeStruct((NP, D), jnp.float32),
)


def _dense2_body(acc_ref, g_ref, d_ref, b_ref, o_ref):
    tot = acc_ref[0:N, :] + acc_ref[NPA:NPA + N, :] + g_ref[0:N, :]
    o_ref[...] = jnp.maximum(_dis_col(d_ref) * tot + b_ref[...], 0.0)


_dense2 = pl.pallas_call(
    _dense2_body,
    out_shape=jax.ShapeDtypeStruct((N, D), jnp.float32),
)


_PAD_IDX = np.arange(EPAD - E, dtype=np.int32) % 16 + N


def kernel(x, edge_index, W, b):
    src = edge_index[0].astype(jnp.int32)
    dst = edge_index[1].astype(jnp.int32)
    padi = jnp.asarray(_PAD_IDX)
    srcp = jnp.concatenate([src, padi])
    dstp = jnp.concatenate([dst, padi])

    src2 = srcp.reshape(NW * CCH, K)
    dst2 = dstp.reshape(NW * CCH, K)

    h = _dense0(x, W)
    degf = _deg_pass(dst2)
    d2 = degf.reshape(NC, NP)

    g = _dense1(h, d2)
    accf = _agg_pass(g, src2, dst2)
    out = _dense2(accf, g, d2, b.reshape(1, D))
    return out


# speedup vs baseline: 43.5558x; 1.0352x over previous
import functools

import jax
import jax.numpy as jnp
import numpy as np
from jax import lax
from jax.experimental import pallas as pl
from jax.experimental.pallas import tpu as pltpu
from jax.experimental.pallas import tpu_sc as plsc

N = 10000
E = 320000
D = 128
NP = 10016
NC = 2
NS = 16
NW = NC * NS
K = 128
CCH = 80
KA = 64
BCH = 16
NB = 10
NBUF = 5
EPW = KA * BCH * NB
EPAD = NW * EPW
NPA = 10240
RPW = NPA // NS

_mesh = plsc.VectorSubcoreMesh(core_axis_name="c", subcore_axis_name="s")


@functools.partial(
    pl.kernel,
    out_type=jax.ShapeDtypeStruct((NC * NP,), jnp.float32),
    mesh=_mesh,
    scratch_types=[
        pltpu.VMEM((CCH, K), jnp.int32),
        pltpu.VMEM((K,), jnp.float32),
        pltpu.VMEM((NP,), jnp.float32),
        pltpu.VMEM_SHARED((NP,), jnp.float32),
        pltpu.SemaphoreType.DMA,
    ],
)
def _deg_pass(dst_hbm, out_hbm, dst_v, ones_v, zero_v, acc_sh, dsem):
    c = lax.axis_index("c")
    s = lax.axis_index("s")
    wid = c * NS + s
    pltpu.sync_copy(dst_hbm.at[pl.ds(wid * CCH, CCH)], dst_v)
    for i in range(K // 16):
        ones_v[pl.ds(16 * i, 16)] = jnp.ones((16,), jnp.float32)

    @pl.when(s == 0)
    def _zero():
        def zbody(i, carry):
            zero_v[pl.ds(i * 16, 16)] = jnp.zeros((16,), jnp.float32)
            return carry

        lax.fori_loop(0, NP // 16, zbody, 0)
        pltpu.sync_copy(zero_v, acc_sh)

    plsc.subcore_barrier()

    def fire(j, carry):
        pltpu.async_copy(ones_v, acc_sh.at[dst_v.at[j]], dsem, add=True)
        return carry

    lax.fori_loop(0, CCH, fire, 0)

    def drain(j, carry):
        pltpu.make_async_copy(ones_v, acc_sh.at[dst_v.at[j]], dsem).wait()
        return carry

    lax.fori_loop(0, CCH, drain, 0)
    plsc.subcore_barrier()

    @pl.when(s == 0)
    def _writeout():
        pltpu.sync_copy(acc_sh, zero_v)
        pltpu.sync_copy(zero_v, out_hbm.at[pl.ds(c * NP, NP)])


@functools.partial(
    pl.kernel,
    out_type=jax.ShapeDtypeStruct((NC * NPA, D), jnp.float32),
    mesh=_mesh,
    scratch_types=[
        pltpu.VMEM((8, K), jnp.int32),
        pltpu.VMEM((8, K), jnp.int32),
        pltpu.VMEM((BCH, KA), jnp.int32),
        pltpu.VMEM((BCH, KA), jnp.int32),
        pltpu.VMEM((KA, D), jnp.float32),
        pltpu.VMEM((KA, D), jnp.float32),
        pltpu.VMEM((KA, D), jnp.float32),
        pltpu.VMEM((KA, D), jnp.float32),
        pltpu.VMEM((KA, D), jnp.float32),
        pltpu.VMEM_SHARED((NPA, D), jnp.float32),
        pltpu.SemaphoreType.DMA,
        pltpu.SemaphoreType.DMA,
        pltpu.SemaphoreType.DMA,
        pltpu.SemaphoreType.DMA,
        pltpu.SemaphoreType.DMA,
        pltpu.SemaphoreType.DMA,
        pltpu.SemaphoreType.DMA,
        pltpu.SemaphoreType.DMA,
        pltpu.SemaphoreType.DMA,
        pltpu.SemaphoreType.DMA,
    ],
)
def _agg_pass(g_hbm, src_hbm, dst_hbm, out_hbm, srcr, dstr, srcb, dstb,
              rows0, rows1, rows2, rows3, rows4, acc_sh, gs0, gs1, gs2, gs3,
              gs4, ss0, ss1, ss2, ss3, ss4):
    c = lax.axis_index("c")
    s = lax.axis_index("s")
    wid = c * NS + s
    rows = (rows0, rows1, rows2, rows3, rows4)
    gs = (gs0, gs1, gs2, gs3, gs4)
    ss = (ss0, ss1, ss2, ss3, ss4)

    def zbody(i, carry):
        for jj in range(D // 16):
            rows0[i, pl.ds(jj * 16, 16)] = jnp.zeros((16,), jnp.float32)
            rows1[i, pl.ds(jj * 16, 16)] = jnp.zeros((16,), jnp.float32)
        return carry

    lax.fori_loop(0, KA, zbody, 0)
    for t in range(RPW // KA // 2):
        pltpu.sync_copy(rows0, acc_sh.at[pl.ds(s * RPW + (2 * t) * KA, KA)])
        pltpu.sync_copy(rows1,
                        acc_sh.at[pl.ds(s * RPW + (2 * t + 1) * KA, KA)])
    plsc.subcore_barrier()

    def _g(b, buf):
        return pltpu.async_copy(g_hbm.at[srcb.at[b]], rows[buf], gs[buf])

    def _wg(b, buf):
        pltpu.make_async_copy(g_hbm.at[srcb.at[b]], rows[buf],
                              gs[buf]).wait()

    def _s(b, buf):
        return pltpu.async_copy(rows[buf], acc_sh.at[dstb.at[b]], ss[buf],
                                add=True)

    def _ws(b, buf):
        pltpu.make_async_copy(rows[buf], acc_sh.at[dstb.at[b]],
                              ss[buf]).wait()

    def block(i, carry):
        base = (wid * NB + i) * 8
        pltpu.sync_copy(src_hbm.at[pl.ds(base, 8)], srcr)
        pltpu.sync_copy(dst_hbm.at[pl.ds(base, 8)], dstr)
        for ch in range(BCH):
            r, hh = ch // 2, ch % 2
            for q in range(KA // 16):
                srcb[ch, pl.ds(16 * q, 16)] = srcr[r,
                                                   pl.ds(64 * hh + 16 * q, 16)]
                dstb[ch, pl.ds(16 * q, 16)] = dstr[r,
                                                   pl.ds(64 * hh + 16 * q, 16)]
        for p in range(NBUF - 1):
            _g(p, p)
        for b in range(BCH):
            _wg(b, b % NBUF)
            _s(b, b % NBUF)
            nb = b + NBUF - 1
            if nb < BCH:
                if b >= 1:
                    _ws(b - 1, (b - 1) % NBUF)
                _g(nb, nb % NBUF)
        for b in range(BCH - NBUF, BCH):
            _ws(b, b % NBUF)
        return carry

    lax.fori_loop(0, NB, block, 0)
    plsc.subcore_barrier()
    pltpu.sync_copy(acc_sh.at[pl.ds(s * RPW, RPW)],
                    out_hbm.at[pl.ds(c * NPA + s * RPW, RPW)])


def _dense0_body(x_ref, w_ref, h_ref):
    h_ref[...] = lax.dot_general(x_ref[...], w_ref[...],
                                 (((1,), (1,)), ((), ())),
                                 precision=lax.Precision.HIGHEST,
                                 preferred_element_type=jnp.float32)


_dense0 = pl.pallas_call(
    _dense0_body,
    out_shape=jax.ShapeDtypeStruct((N, D), jnp.float32),
)


def _dis_col(d_ref):
    deg = d_ref[0:1, 0:N] + d_ref[1:2, 0:N] + 1.0
    return lax.transpose(lax.rsqrt(deg), (1, 0))


def _dense1_body(h_ref, d_ref, g_ref):
    g_ref[0:N, :] = _dis_col(d_ref) * h_ref[...]
    g_ref[N:NP, :] = jnp.zeros((NP - N, D), jnp.float32)


_dense1 = pl.pallas_call(
    _dense1_body,
    out_shape=jax.ShapeDtypeStruct((NP, D), jnp.float32),
)


def _dense2_body(acc_ref, g_ref, d_ref, b_ref, o_ref):
    tot = acc_ref[0:N, :] + acc_ref[NPA:NPA + N, :] + g_ref[0:N, :]
    o_ref[...] = jnp.maximum(_dis_col(d_ref) * tot + b_ref[...], 0.0)


_dense2 = pl.pallas_call(
    _dense2_body,
    out_shape=jax.ShapeDtypeStruct((N, D), jnp.float32),
)


_PAD_IDX = np.arange(EPAD - E, dtype=np.int32) % 16 + N


def kernel(x, edge_index, W, b):
    src = edge_index[0].astype(jnp.int32)
    dst = edge_index[1].astype(jnp.int32)
    padi = jnp.asarray(_PAD_IDX)
    srcp = jnp.concatenate([src, padi])
    dstp = jnp.concatenate([dst, padi])

    src2 = srcp.reshape(NW * CCH, K)
    dst2 = dstp.reshape(NW * CCH, K)

    h = _dense0(x, W)
    degf = _deg_pass(dst2)
    d2 = degf.reshape(NC, NP)

    g = _dense1(h, d2)
    accf = _agg_pass(g, src2, dst2)
    out = _dense2(accf, g, d2, b.reshape(1, D))
    return out


# speedup vs baseline: 46.1525x; 1.0596x over previous
import functools

import jax
import jax.numpy as jnp
import numpy as np
from jax import lax
from jax.experimental import pallas as pl
from jax.experimental.pallas import tpu as pltpu
from jax.experimental.pallas import tpu_sc as plsc

N = 10000
E = 320000
D = 128
NP = 10016
NC = 2
NS = 16
NW = NC * NS
K = 128
EPT = E // NW
CDF = EPT // K
KA = 64
BCH = 16
BE = KA * BCH
NB = 9
TFC = 12
TR = EPT - NB * BE - TFC * KA
NBUF = 5
NPA = 10240
RPW = NPA // NS

_mesh = plsc.VectorSubcoreMesh(core_axis_name="c", subcore_axis_name="s")


@functools.partial(
    pl.kernel,
    out_type=jax.ShapeDtypeStruct((NC * NP,), jnp.float32),
    mesh=_mesh,
    scratch_types=[
        pltpu.VMEM((CDF + 1, K), jnp.int32),
        pltpu.VMEM((EPT,), jnp.int32),
        pltpu.VMEM((K,), jnp.float32),
        pltpu.VMEM((NP,), jnp.float32),
        pltpu.VMEM_SHARED((NP,), jnp.float32),
        pltpu.SemaphoreType.DMA,
    ],
)
def _deg_pass(dst_hbm, out_hbm, dst_v, stage, ones_v, zero_v, acc_sh, dsem):
    c = lax.axis_index("c")
    s = lax.axis_index("s")
    wid = c * NS + s
    pltpu.sync_copy(dst_hbm.at[pl.ds(wid * EPT, EPT)], stage)
    for i in range(K // 16):
        ones_v[pl.ds(16 * i, 16)] = jnp.ones((16,), jnp.float32)

    def rbody(r, carry):
        for q in range(K // 16):
            dst_v[r, pl.ds(16 * q, 16)] = stage[pl.ds(r * K + 16 * q, 16)]
        return carry

    lax.fori_loop(0, CDF, rbody, 0)
    dst_v[CDF, pl.ds(0, 16)] = stage[pl.ds(CDF * K, 16)]
    padv = lax.iota(jnp.int32, 16) + N
    for q in range(1, K // 16):
        dst_v[CDF, pl.ds(16 * q, 16)] = padv

    @pl.when(s == 0)
    def _zero():
        def zbody(i, carry):
            zero_v[pl.ds(i * 16, 16)] = jnp.zeros((16,), jnp.float32)
            return carry

        lax.fori_loop(0, NP // 16, zbody, 0)
        pltpu.sync_copy(zero_v, acc_sh)

    plsc.subcore_barrier()

    def fire(j, carry):
        pltpu.async_copy(ones_v, acc_sh.at[dst_v.at[j]], dsem, add=True)
        return carry

    lax.fori_loop(0, CDF + 1, fire, 0)

    def drain(j, carry):
        pltpu.make_async_copy(ones_v, acc_sh.at[dst_v.at[j]], dsem).wait()
        return carry

    lax.fori_loop(0, CDF + 1, drain, 0)
    plsc.subcore_barrier()

    @pl.when(s == 0)
    def _writeout():
        pltpu.sync_copy(acc_sh, zero_v)
        pltpu.sync_copy(zero_v, out_hbm.at[pl.ds(c * NP, NP)])


@functools.partial(
    pl.kernel,
    out_type=jax.ShapeDtypeStruct((NC * NPA, D), jnp.float32),
    mesh=_mesh,
    scratch_types=[
        pltpu.VMEM((BE,), jnp.int32),
        pltpu.VMEM((BE,), jnp.int32),
        pltpu.VMEM((BCH, KA), jnp.int32),
        pltpu.VMEM((BCH, KA), jnp.int32),
        pltpu.VMEM((KA, D), jnp.float32),
        pltpu.VMEM((KA, D), jnp.float32),
        pltpu.VMEM((KA, D), jnp.float32),
        pltpu.VMEM((KA, D), jnp.float32),
        pltpu.VMEM((KA, D), jnp.float32),
        pltpu.VMEM_SHARED((NPA, D), jnp.float32),
        pltpu.SemaphoreType.DMA,
        pltpu.SemaphoreType.DMA,
        pltpu.SemaphoreType.DMA,
        pltpu.SemaphoreType.DMA,
        pltpu.SemaphoreType.DMA,
        pltpu.SemaphoreType.DMA,
        pltpu.SemaphoreType.DMA,
        pltpu.SemaphoreType.DMA,
        pltpu.SemaphoreType.DMA,
        pltpu.SemaphoreType.DMA,
        pltpu.SemaphoreType.DMA,
    ],
)
def _agg_pass(g_hbm, src_hbm, dst_hbm, out_hbm, srcr, dstr, srcb, dstb,
              rows0, rows1, rows2, rows3, rows4, acc_sh, gs0, gs1, gs2, gs3,
              gs4, ss0, ss1, ss2, ss3, ss4, isem):
    c = lax.axis_index("c")
    s = lax.axis_index("s")
    wid = c * NS + s
    rows = (rows0, rows1, rows2, rows3, rows4)
    gs = (gs0, gs1, gs2, gs3, gs4)
    ss = (ss0, ss1, ss2, ss3, ss4)

    def zbody(i, carry):
        for jj in range(D // 16):
            rows0[i, pl.ds(jj * 16, 16)] = jnp.zeros((16,), jnp.float32)
            rows1[i, pl.ds(jj * 16, 16)] = jnp.zeros((16,), jnp.float32)
        return carry

    lax.fori_loop(0, KA, zbody, 0)
    for t in range(RPW // KA // 2):
        pltpu.sync_copy(rows0, acc_sh.at[pl.ds(s * RPW + (2 * t) * KA, KA)])
        pltpu.sync_copy(rows1,
                        acc_sh.at[pl.ds(s * RPW + (2 * t + 1) * KA, KA)])
    plsc.subcore_barrier()

    def _g(b, buf):
        return pltpu.async_copy(g_hbm.at[srcb.at[b]], rows[buf], gs[buf])

    def _wg(b, buf):
        pltpu.make_async_copy(g_hbm.at[srcb.at[b]], rows[buf],
                              gs[buf]).wait()

    def _s(b, buf):
        return pltpu.async_copy(rows[buf], acc_sh.at[dstb.at[b]], ss[buf],
                                add=True)

    def _ws(b, buf):
        pltpu.make_async_copy(rows[buf], acc_sh.at[dstb.at[b]],
                              ss[buf]).wait()

    def _pipeline(nch):
        for p in range(NBUF - 1):
            _g(p, p)
        for b in range(nch):
            _wg(b, b % NBUF)
            _s(b, b % NBUF)
            nb = b + NBUF - 1
            if nb < nch:
                if b >= 1:
                    _ws(b - 1, (b - 1) % NBUF)
                _g(nb, nb % NBUF)
        for b in range(max(0, nch - NBUF), nch):
            _ws(b, b % NBUF)

    def _repack(nch):
        for ch in range(nch):
            for q in range(KA // 16):
                o = KA * ch + 16 * q
                srcb[ch, pl.ds(16 * q, 16)] = srcr[pl.ds(o, 16)]
                dstb[ch, pl.ds(16 * q, 16)] = dstr[pl.ds(o, 16)]

    e0 = wid * EPT

    def _fire_idx(base, n):
        pltpu.async_copy(src_hbm.at[pl.ds(base, n)], srcr.at[pl.ds(0, n)],
                         isem)
        pltpu.async_copy(dst_hbm.at[pl.ds(base, n)], dstr.at[pl.ds(0, n)],
                         isem)

    def _wait_idx(base, n):
        pltpu.make_async_copy(src_hbm.at[pl.ds(base, n)],
                              srcr.at[pl.ds(0, n)], isem).wait()
        pltpu.make_async_copy(dst_hbm.at[pl.ds(base, n)],
                              dstr.at[pl.ds(0, n)], isem).wait()

    _fire_idx(e0, BE)

    def block(i, carry):
        base = e0 + i * BE
        _wait_idx(base, BE)
        _repack(BCH)
        nxt = e0 + jnp.minimum(i + 1, NB - 1) * BE
        _fire_idx(nxt, BE)
        _pipeline(BCH)
        return carry

    lax.fori_loop(0, NB, block, 0)

    _wait_idx(0, BE)
    tbase = e0 + NB * BE
    tn = TFC * KA + TR
    pltpu.sync_copy(src_hbm.at[pl.ds(tbase, tn)], srcr.at[pl.ds(0, tn)])
    pltpu.sync_copy(dst_hbm.at[pl.ds(tbase, tn)], dstr.at[pl.ds(0, tn)])
    _repack(TFC)
    srcb[TFC, pl.ds(0, 16)] = srcr[pl.ds(TFC * KA, 16)]
    dstb[TFC, pl.ds(0, 16)] = dstr[pl.ds(TFC * KA, 16)]
    padv = lax.iota(jnp.int32, 16) + N
    for q in range(1, KA // 16):
        srcb[TFC, pl.ds(16 * q, 16)] = padv
        dstb[TFC, pl.ds(16 * q, 16)] = padv
    _pipeline(TFC + 1)
    plsc.subcore_barrier()
    pltpu.sync_copy(acc_sh.at[pl.ds(s * RPW, RPW)],
                    out_hbm.at[pl.ds(c * NPA + s * RPW, RPW)])


def _dense0_body(x_ref, w_ref, h_ref):
    h_ref[...] = lax.dot_general(x_ref[...], w_ref[...],
                                 (((1,), (1,)), ((), ())),
                                 precision=lax.Precision.HIGHEST,
                                 preferred_element_type=jnp.float32)


_dense0 = pl.pallas_call(
    _dense0_body,
    out_shape=jax.ShapeDtypeStruct((N, D), jnp.float32),
)


def _dis_col(d_ref):
    deg = d_ref[0:1, 0:N] + d_ref[1:2, 0:N] + 1.0
    return lax.transpose(lax.rsqrt(deg), (1, 0))


def _dense1_body(h_ref, d_ref, g_ref):
    g_ref[0:N, :] = _dis_col(d_ref) * h_ref[...]
    g_ref[N:NP, :] = jnp.zeros((NP - N, D), jnp.float32)


_dense1 = pl.pallas_call(
    _dense1_body,
    out_shape=jax.ShapeDtypeStruct((NP, D), jnp.float32),
)


def _dense2_body(acc_ref, g_ref, d_ref, b_ref, o_ref):
    tot = acc_ref[0:N, :] + acc_ref[NPA:NPA + N, :] + g_ref[0:N, :]
    o_ref[...] = jnp.maximum(_dis_col(d_ref) * tot + b_ref[...], 0.0)


_dense2 = pl.pallas_call(
    _dense2_body,
    out_shape=jax.ShapeDtypeStruct((N, D), jnp.float32),
)


def kernel(x, edge_index, W, b):
    ei = edge_index.astype(jnp.int32)
    src = ei[0]
    dst = ei[1]

    h = _dense0(x, W)
    degf = _deg_pass(dst)
    d2 = degf.reshape(NC, NP)

    g = _dense1(h, d2)
    accf = _agg_pass(g, src, dst)
    out = _dense2(accf, g, d2, b.reshape(1, D))
    return out


# speedup vs baseline: 49.3852x; 1.0700x over previous
import functools

import jax
import jax.numpy as jnp
import numpy as np
from jax import lax
from jax.experimental import pallas as pl
from jax.experimental.pallas import tpu as pltpu
from jax.experimental.pallas import tpu_sc as plsc

N = 10000
E = 320000
D = 128
NP = 10016
NC = 2
NS = 16
NW = NC * NS
K = 128
ER = E // K
RHI = 79
RLO = 78
NHI = 4
KA = 64
BCH = 16
BE = KA * BCH
NB = 9
NBUF = 5
NPA = 10240
RPW = NPA // NS

_mesh = plsc.VectorSubcoreMesh(core_axis_name="c", subcore_axis_name="s")


@functools.partial(
    pl.kernel,
    out_type=jax.ShapeDtypeStruct((NC * NP,), jnp.float32),
    mesh=_mesh,
    scratch_types=[
        pltpu.VMEM((RHI, K), jnp.int32),
        pltpu.VMEM((2, RHI * K), jnp.int32),
        pltpu.VMEM((K,), jnp.float32),
        pltpu.VMEM((NP,), jnp.float32),
        pltpu.VMEM_SHARED((NP,), jnp.float32),
        pltpu.SemaphoreType.DMA,
    ],
)
def _deg_pass(ei_hbm, out_hbm, dst_v, stage, ones_v, zero_v, acc_sh, dsem):
    c = lax.axis_index("c")
    s = lax.axis_index("s")
    wid = c * NS + s
    hi = wid < NHI
    e0 = jnp.where(hi, wid * (RHI * K), NHI * RHI * K + (wid - NHI) * (RLO * K))
    nch = jnp.where(hi, RHI, RLO)
    pltpu.sync_copy(ei_hbm.at[pl.ds(0, 2), pl.ds(e0, RLO * K)],
                    stage.at[pl.ds(0, 2), pl.ds(0, RLO * K)])

    @pl.when(hi)
    def _extra():
        pltpu.sync_copy(ei_hbm.at[pl.ds(0, 2), pl.ds(e0 + RLO * K, K)],
                        stage.at[pl.ds(0, 2), pl.ds(RLO * K, K)])

    for i in range(K // 16):
        ones_v[pl.ds(16 * i, 16)] = jnp.ones((16,), jnp.float32)

    def rbody(r, carry):
        for q in range(K // 16):
            dst_v[r, pl.ds(16 * q, 16)] = stage[1, pl.ds(r * K + 16 * q, 16)]
        return carry

    lax.fori_loop(0, nch, rbody, 0)

    @pl.when(s == 0)
    def _zero():
        def zbody(i, carry):
            zero_v[pl.ds(i * 16, 16)] = jnp.zeros((16,), jnp.float32)
            return carry

        lax.fori_loop(0, NP // 16, zbody, 0)
        pltpu.sync_copy(zero_v, acc_sh)

    plsc.subcore_barrier()

    def fire(j, carry):
        pltpu.async_copy(ones_v, acc_sh.at[dst_v.at[j]], dsem, add=True)
        return carry

    lax.fori_loop(0, nch, fire, 0)

    def drain(j, carry):
        pltpu.make_async_copy(ones_v, acc_sh.at[dst_v.at[j]], dsem).wait()
        return carry

    lax.fori_loop(0, nch, drain, 0)
    plsc.subcore_barrier()

    @pl.when(s == 0)
    def _writeout():
        pltpu.sync_copy(acc_sh, zero_v)
        pltpu.sync_copy(zero_v, out_hbm.at[pl.ds(c * NP, NP)])


@functools.partial(
    pl.kernel,
    out_type=jax.ShapeDtypeStruct((NC * NPA, D), jnp.float32),
    mesh=_mesh,
    scratch_types=[
        pltpu.VMEM((2, BE), jnp.int32),
        pltpu.VMEM((BCH, KA), jnp.int32),
        pltpu.VMEM((BCH, KA), jnp.int32),
        pltpu.VMEM((KA, D), jnp.float32),
        pltpu.VMEM((KA, D), jnp.float32),
        pltpu.VMEM((KA, D), jnp.float32),
        pltpu.VMEM((KA, D), jnp.float32),
        pltpu.VMEM((KA, D), jnp.float32),
        pltpu.VMEM_SHARED((NPA, D), jnp.float32),
        pltpu.SemaphoreType.DMA,
        pltpu.SemaphoreType.DMA,
        pltpu.SemaphoreType.DMA,
        pltpu.SemaphoreType.DMA,
        pltpu.SemaphoreType.DMA,
        pltpu.SemaphoreType.DMA,
        pltpu.SemaphoreType.DMA,
        pltpu.SemaphoreType.DMA,
        pltpu.SemaphoreType.DMA,
        pltpu.SemaphoreType.DMA,
        pltpu.SemaphoreType.DMA,
    ],
)
def _agg_pass(g_hbm, ei_hbm, out_hbm, stage, srcb, dstb,
              rows0, rows1, rows2, rows3, rows4, acc_sh, gs0, gs1, gs2, gs3,
              gs4, ss0, ss1, ss2, ss3, ss4, isem):
    c = lax.axis_index("c")
    s = lax.axis_index("s")
    wid = c * NS + s
    rows = (rows0, rows1, rows2, rows3, rows4)
    gs = (gs0, gs1, gs2, gs3, gs4)
    ss = (ss0, ss1, ss2, ss3, ss4)

    def zbody(i, carry):
        for jj in range(D // 16):
            rows0[i, pl.ds(jj * 16, 16)] = jnp.zeros((16,), jnp.float32)
            rows1[i, pl.ds(jj * 16, 16)] = jnp.zeros((16,), jnp.float32)
        return carry

    lax.fori_loop(0, KA, zbody, 0)
    for t in range(RPW // KA // 2):
        pltpu.sync_copy(rows0, acc_sh.at[pl.ds(s * RPW + (2 * t) * KA, KA)])
        pltpu.sync_copy(rows1,
                        acc_sh.at[pl.ds(s * RPW + (2 * t + 1) * KA, KA)])
    plsc.subcore_barrier()

    def _g(b, buf):
        return pltpu.async_copy(g_hbm.at[srcb.at[b]], rows[buf], gs[buf])

    def _wg(b, buf):
        pltpu.make_async_copy(g_hbm.at[srcb.at[b]], rows[buf],
                              gs[buf]).wait()

    def _s(b, buf):
        return pltpu.async_copy(rows[buf], acc_sh.at[dstb.at[b]], ss[buf],
                                add=True)

    def _ws(b, buf):
        pltpu.make_async_copy(rows[buf], acc_sh.at[dstb.at[b]],
                              ss[buf]).wait()

    def _pipeline(nch):
        for p in range(NBUF - 1):
            _g(p, p)
        for b in range(nch):
            _wg(b, b % NBUF)
            _s(b, b % NBUF)
            nb = b + NBUF - 1
            if nb < nch:
                if b >= 1:
                    _ws(b - 1, (b - 1) % NBUF)
                _g(nb, nb % NBUF)
        for b in range(max(0, nch - NBUF), nch):
            _ws(b, b % NBUF)

    def _repack(nch):
        for ch in range(nch):
            for q in range(KA // 16):
                o = KA * ch + 16 * q
                srcb[ch, pl.ds(16 * q, 16)] = stage[0, pl.ds(o, 16)]
                dstb[ch, pl.ds(16 * q, 16)] = stage[1, pl.ds(o, 16)]

    hi = wid < NHI
    e0 = jnp.where(hi, wid * (RHI * K), NHI * RHI * K + (wid - NHI) * (RLO * K))

    def _fire_idx(base):
        pltpu.async_copy(ei_hbm.at[pl.ds(0, 2), pl.ds(base, BE)], stage,
                         isem)

    def _wait_idx():
        pltpu.make_async_copy(ei_hbm.at[pl.ds(0, 2), pl.ds(0, BE)], stage,
                              isem).wait()

    _fire_idx(e0)

    def block(i, carry):
        _wait_idx()
        _repack(BCH)
        nxt = jnp.minimum(e0 + (i + 1) * BE, E - BE)
        _fire_idx(nxt)
        _pipeline(BCH)
        return carry

    lax.fori_loop(0, NB, block, 0)

    _wait_idx()
    tbase = e0 + NB * BE
    pltpu.sync_copy(ei_hbm.at[pl.ds(0, 2), pl.ds(tbase, (RLO - 72) * K)],
                    stage.at[pl.ds(0, 2), pl.ds(0, (RLO - 72) * K)])

    @pl.when(hi)
    def _tail_hi():
        pltpu.sync_copy(ei_hbm.at[pl.ds(0, 2), pl.ds(tbase + (RLO - 72) * K, K)],
                        stage.at[pl.ds(0, 2), pl.ds((RLO - 72) * K, K)])
        _repack((RHI - 72) * 2)
        _pipeline((RHI - 72) * 2)

    @pl.when(jnp.logical_not(hi))
    def _tail_lo():
        _repack((RLO - 72) * 2)
        _pipeline((RLO - 72) * 2)
    plsc.subcore_barrier()
    pltpu.sync_copy(acc_sh.at[pl.ds(s * RPW, RPW)],
                    out_hbm.at[pl.ds(c * NPA + s * RPW, RPW)])


def _dense0_body(x_ref, w_ref, h_ref):
    h_ref[...] = lax.dot_general(x_ref[...], w_ref[...],
                                 (((1,), (1,)), ((), ())),
                                 precision=lax.Precision.HIGHEST,
                                 preferred_element_type=jnp.float32)


_dense0 = pl.pallas_call(
    _dense0_body,
    out_shape=jax.ShapeDtypeStruct((N, D), jnp.float32),
)


def _dis_col(d_ref):
    deg = d_ref[0:1, 0:N] + d_ref[1:2, 0:N] + 1.0
    return lax.transpose(lax.rsqrt(deg), (1, 0))


def _dense1_body(h_ref, d_ref, g_ref):
    g_ref[0:N, :] = _dis_col(d_ref) * h_ref[...]
    g_ref[N:NP, :] = jnp.zeros((NP - N, D), jnp.float32)


_dense1 = pl.pallas_call(
    _dense1_body,
    out_shape=jax.ShapeDtypeStruct((NP, D), jnp.float32),
)


def _dense2_body(acc_ref, g_ref, d_ref, b_ref, o_ref):
    tot = acc_ref[0:N, :] + acc_ref[NPA:NPA + N, :] + g_ref[0:N, :]
    o_ref[...] = jnp.maximum(_dis_col(d_ref) * tot + b_ref[...], 0.0)


_dense2 = pl.pallas_call(
    _dense2_body,
    out_shape=jax.ShapeDtypeStruct((N, D), jnp.float32),
)


def kernel(x, edge_index, W, b):
    ei = edge_index.astype(jnp.int32)

    h = _dense0(x, W)
    degf = _deg_pass(ei)
    d2 = degf.reshape(NC, NP)

    g = _dense1(h, d2)
    accf = _agg_pass(g, ei)
    out = _dense2(accf, g, d2, b.reshape(1, D))
    return out


# speedup vs baseline: 50.6003x; 1.0246x over previous
import functools

import jax
import jax.numpy as jnp
import numpy as np
from jax import lax
from jax.experimental import pallas as pl
from jax.experimental.pallas import tpu as pltpu
from jax.experimental.pallas import tpu_sc as plsc

N = 10000
E = 320000
D = 128
NP = 10240
NC = 2
NS = 16
NW = NC * NS
K = 128
ER = E // K
RHI = 79
RLO = 78
NHI = 4
KA = 64
BCH = 16
BE = KA * BCH
NB = 9
NBUF = 5
NPA = 10240
RPW = NPA // NS

_mesh = plsc.VectorSubcoreMesh(core_axis_name="c", subcore_axis_name="s")


@functools.partial(
    pl.kernel,
    out_type=jax.ShapeDtypeStruct((NC * NP,), jnp.float32),
    mesh=_mesh,
    scratch_types=[
        pltpu.VMEM((RHI, K), jnp.int32),
        pltpu.VMEM((2, RHI * K), jnp.int32),
        pltpu.VMEM((K,), jnp.float32),
        pltpu.VMEM((NP // NS,), jnp.float32),
        pltpu.VMEM_SHARED((NP,), jnp.float32),
        pltpu.SemaphoreType.DMA,
    ],
)
def _deg_pass(ei_hbm, out_hbm, dst_v, stage, ones_v, zero_v, acc_sh, dsem):
    c = lax.axis_index("c")
    s = lax.axis_index("s")
    wid = c * NS + s
    hi = wid < NHI
    e0 = jnp.where(hi, wid * (RHI * K), NHI * RHI * K + (wid - NHI) * (RLO * K))
    nch = jnp.where(hi, RHI, RLO)
    pltpu.sync_copy(ei_hbm.at[pl.ds(0, 2), pl.ds(e0, RLO * K)],
                    stage.at[pl.ds(0, 2), pl.ds(0, RLO * K)])

    @pl.when(hi)
    def _extra():
        pltpu.sync_copy(ei_hbm.at[pl.ds(0, 2), pl.ds(e0 + RLO * K, K)],
                        stage.at[pl.ds(0, 2), pl.ds(RLO * K, K)])

    for i in range(K // 16):
        ones_v[pl.ds(16 * i, 16)] = jnp.ones((16,), jnp.float32)

    def rbody(r, carry):
        for q in range(K // 16):
            dst_v[r, pl.ds(16 * q, 16)] = stage[1, pl.ds(r * K + 16 * q, 16)]
        return carry

    lax.fori_loop(0, nch, rbody, 0)

    def zbody(i, carry):
        zero_v[pl.ds(i * 16, 16)] = jnp.zeros((16,), jnp.float32)
        return carry

    lax.fori_loop(0, NP // NS // 16, zbody, 0)
    pltpu.sync_copy(zero_v, acc_sh.at[pl.ds(s * (NP // NS), NP // NS)])
    plsc.subcore_barrier()

    def fire(j, carry):
        pltpu.async_copy(ones_v, acc_sh.at[dst_v.at[j]], dsem, add=True)
        return carry

    lax.fori_loop(0, nch, fire, 0)

    def drain(j, carry):
        pltpu.make_async_copy(ones_v, acc_sh.at[dst_v.at[j]], dsem).wait()
        return carry

    lax.fori_loop(0, nch, drain, 0)
    plsc.subcore_barrier()
    pltpu.sync_copy(acc_sh.at[pl.ds(s * (NP // NS), NP // NS)], zero_v)
    pltpu.sync_copy(zero_v,
                    out_hbm.at[pl.ds(c * NP + s * (NP // NS), NP // NS)])


@functools.partial(
    pl.kernel,
    out_type=jax.ShapeDtypeStruct((NC * NPA, D), jnp.float32),
    mesh=_mesh,
    scratch_types=[
        pltpu.VMEM((2, BE), jnp.int32),
        pltpu.VMEM((BCH, KA), jnp.int32),
        pltpu.VMEM((BCH, KA), jnp.int32),
        pltpu.VMEM((KA, D), jnp.float32),
        pltpu.VMEM((KA, D), jnp.float32),
        pltpu.VMEM((KA, D), jnp.float32),
        pltpu.VMEM((KA, D), jnp.float32),
        pltpu.VMEM((KA, D), jnp.float32),
        pltpu.VMEM_SHARED((NPA, D), jnp.float32),
        pltpu.SemaphoreType.DMA,
        pltpu.SemaphoreType.DMA,
        pltpu.SemaphoreType.DMA,
        pltpu.SemaphoreType.DMA,
        pltpu.SemaphoreType.DMA,
        pltpu.SemaphoreType.DMA,
        pltpu.SemaphoreType.DMA,
        pltpu.SemaphoreType.DMA,
        pltpu.SemaphoreType.DMA,
        pltpu.SemaphoreType.DMA,
        pltpu.SemaphoreType.DMA,
    ],
)
def _agg_pass(g_hbm, ei_hbm, out_hbm, stage, srcb, dstb,
              rows0, rows1, rows2, rows3, rows4, acc_sh, gs0, gs1, gs2, gs3,
              gs4, ss0, ss1, ss2, ss3, ss4, isem):
    c = lax.axis_index("c")
    s = lax.axis_index("s")
    wid = c * NS + s
    rows = (rows0, rows1, rows2, rows3, rows4)
    gs = (gs0, gs1, gs2, gs3, gs4)
    ss = (ss0, ss1, ss2, ss3, ss4)

    def zbody(i, carry):
        for jj in range(D // 16):
            rows0[i, pl.ds(jj * 16, 16)] = jnp.zeros((16,), jnp.float32)
            rows1[i, pl.ds(jj * 16, 16)] = jnp.zeros((16,), jnp.float32)
        return carry

    lax.fori_loop(0, KA, zbody, 0)
    for t in range(RPW // KA // 2):
        pltpu.sync_copy(rows0, acc_sh.at[pl.ds(s * RPW + (2 * t) * KA, KA)])
        pltpu.sync_copy(rows1,
                        acc_sh.at[pl.ds(s * RPW + (2 * t + 1) * KA, KA)])
    plsc.subcore_barrier()

    def _g(b, buf):
        return pltpu.async_copy(g_hbm.at[srcb.at[b]], rows[buf], gs[buf])

    def _wg(b, buf):
        pltpu.make_async_copy(g_hbm.at[srcb.at[b]], rows[buf],
                              gs[buf]).wait()

    def _s(b, buf):
        return pltpu.async_copy(rows[buf], acc_sh.at[dstb.at[b]], ss[buf],
                                add=True)

    def _ws(b, buf):
        pltpu.make_async_copy(rows[buf], acc_sh.at[dstb.at[b]],
                              ss[buf]).wait()

    def _pipeline(nch):
        for p in range(NBUF - 1):
            _g(p, p)
        for b in range(nch):
            _wg(b, b % NBUF)
            _s(b, b % NBUF)
            nb = b + NBUF - 1
            if nb < nch:
                if b >= 1:
                    _ws(b - 1, (b - 1) % NBUF)
                _g(nb, nb % NBUF)
        for b in range(max(0, nch - NBUF), nch):
            _ws(b, b % NBUF)

    def _repack(nch):
        for ch in range(nch):
            for q in range(KA // 16):
                o = KA * ch + 16 * q
                srcb[ch, pl.ds(16 * q, 16)] = stage[0, pl.ds(o, 16)]
                dstb[ch, pl.ds(16 * q, 16)] = stage[1, pl.ds(o, 16)]

    hi = wid < NHI
    e0 = jnp.where(hi, wid * (RHI * K), NHI * RHI * K + (wid - NHI) * (RLO * K))

    def _fire_idx(base):
        pltpu.async_copy(ei_hbm.at[pl.ds(0, 2), pl.ds(base, BE)], stage,
                         isem)

    def _wait_idx():
        pltpu.make_async_copy(ei_hbm.at[pl.ds(0, 2), pl.ds(0, BE)], stage,
                              isem).wait()

    _fire_idx(e0)

    def block(i, carry):
        _wait_idx()
        _repack(BCH)
        nxt = jnp.minimum(e0 + (i + 1) * BE, E - BE)
        _fire_idx(nxt)
        _pipeline(BCH)
        return carry

    lax.fori_loop(0, NB, block, 0)

    _wait_idx()
    tbase = e0 + NB * BE
    pltpu.sync_copy(ei_hbm.at[pl.ds(0, 2), pl.ds(tbase, (RLO - 72) * K)],
                    stage.at[pl.ds(0, 2), pl.ds(0, (RLO - 72) * K)])

    @pl.when(hi)
    def _tail_hi():
        pltpu.sync_copy(ei_hbm.at[pl.ds(0, 2), pl.ds(tbase + (RLO - 72) * K, K)],
                        stage.at[pl.ds(0, 2), pl.ds((RLO - 72) * K, K)])
        _repack((RHI - 72) * 2)
        _pipeline((RHI - 72) * 2)

    @pl.when(jnp.logical_not(hi))
    def _tail_lo():
        _repack((RLO - 72) * 2)
        _pipeline((RLO - 72) * 2)
    plsc.subcore_barrier()
    pltpu.sync_copy(acc_sh.at[pl.ds(s * RPW, RPW)],
                    out_hbm.at[pl.ds(c * NPA + s * RPW, RPW)])


def _dense0_body(x_ref, w_ref, h_ref):
    h_ref[...] = lax.dot_general(x_ref[...], w_ref[...],
                                 (((1,), (1,)), ((), ())),
                                 precision=lax.Precision.HIGHEST,
                                 preferred_element_type=jnp.float32)


_dense0 = pl.pallas_call(
    _dense0_body,
    out_shape=jax.ShapeDtypeStruct((N, D), jnp.float32),
)


def _dis_col(d_ref):
    deg = d_ref[0:1, 0:N] + d_ref[1:2, 0:N] + 1.0
    return lax.transpose(lax.rsqrt(deg), (1, 0))


def _dense1_body(h_ref, d_ref, g_ref):
    g_ref[...] = _dis_col(d_ref) * h_ref[...]


_dense1 = pl.pallas_call(
    _dense1_body,
    out_shape=jax.ShapeDtypeStruct((N, D), jnp.float32),
)


def _dense2_body(acc_ref, g_ref, d_ref, b_ref, o_ref):
    tot = acc_ref[0:N, :] + acc_ref[NPA:NPA + N, :] + g_ref[...]
    o_ref[...] = jnp.maximum(_dis_col(d_ref) * tot + b_ref[...], 0.0)


_dense2 = pl.pallas_call(
    _dense2_body,
    out_shape=jax.ShapeDtypeStruct((N, D), jnp.float32),
)


def kernel(x, edge_index, W, b):
    ei = edge_index.astype(jnp.int32)

    h = _dense0(x, W)
    degf = _deg_pass(ei)
    d2 = degf.reshape(NC, NP)

    g = _dense1(h, d2)
    accf = _agg_pass(g, ei)
    out = _dense2(accf, g, d2, b.reshape(1, D))
    return out


# speedup vs baseline: 50.7108x; 1.0022x over previous
import functools

import jax
import jax.numpy as jnp
import numpy as np
from jax import lax
from jax.experimental import pallas as pl
from jax.experimental.pallas import tpu as pltpu
from jax.experimental.pallas import tpu_sc as plsc

N = 10000
E = 320000
D = 128
NP = 10240
NC = 2
NS = 16
NW = NC * NS
K = 128
ER = E // K
RHI = 79
RLO = 78
NHI = 4
KA = 64
BCH = 16
BE = KA * BCH
NB = 9
NBUF = 5
NPA = 10240
RPW = NPA // NS

_mesh = plsc.VectorSubcoreMesh(core_axis_name="c", subcore_axis_name="s")


@functools.partial(
    pl.kernel,
    out_type=jax.ShapeDtypeStruct((NC * NP,), jnp.float32),
    mesh=_mesh,
    scratch_types=[
        pltpu.VMEM((RHI, K), jnp.int32),
        pltpu.VMEM((2, RHI * K), jnp.int32),
        pltpu.VMEM((K,), jnp.float32),
        pltpu.VMEM((NP // NS,), jnp.float32),
        pltpu.VMEM_SHARED((NP,), jnp.float32),
        pltpu.SemaphoreType.DMA,
    ],
)
def _deg_pass(ei_hbm, out_hbm, dst_v, stage, ones_v, zero_v, acc_sh, dsem):
    c = lax.axis_index("c")
    s = lax.axis_index("s")
    wid = c * NS + s
    hi = wid < NHI
    e0 = jnp.where(hi, wid * (RHI * K), NHI * RHI * K + (wid - NHI) * (RLO * K))
    nch = jnp.where(hi, RHI, RLO)
    pltpu.sync_copy(ei_hbm.at[pl.ds(0, 2), pl.ds(e0, RLO * K)],
                    stage.at[pl.ds(0, 2), pl.ds(0, RLO * K)])

    @pl.when(hi)
    def _extra():
        pltpu.sync_copy(ei_hbm.at[pl.ds(0, 2), pl.ds(e0 + RLO * K, K)],
                        stage.at[pl.ds(0, 2), pl.ds(RLO * K, K)])

    for i in range(K // 16):
        ones_v[pl.ds(16 * i, 16)] = jnp.ones((16,), jnp.float32)

    def rbody(r, carry):
        for q in range(K // 16):
            dst_v[r, pl.ds(16 * q, 16)] = stage[1, pl.ds(r * K + 16 * q, 16)]
        return carry

    lax.fori_loop(0, nch, rbody, 0)

    def zbody(i, carry):
        zero_v[pl.ds(i * 16, 16)] = jnp.zeros((16,), jnp.float32)
        return carry

    lax.fori_loop(0, NP // NS // 16, zbody, 0)
    pltpu.sync_copy(zero_v, acc_sh.at[pl.ds(s * (NP // NS), NP // NS)])
    plsc.subcore_barrier()

    def fire(j, carry):
        pltpu.async_copy(ones_v, acc_sh.at[dst_v.at[j]], dsem, add=True)
        return carry

    lax.fori_loop(0, nch, fire, 0)

    def drain(j, carry):
        pltpu.make_async_copy(ones_v, acc_sh.at[dst_v.at[j]], dsem).wait()
        return carry

    lax.fori_loop(0, nch, drain, 0)
    plsc.subcore_barrier()
    pltpu.sync_copy(acc_sh.at[pl.ds(s * (NP // NS), NP // NS)], zero_v)
    pltpu.sync_copy(zero_v,
                    out_hbm.at[pl.ds(c * NP + s * (NP // NS), NP // NS)])


@functools.partial(
    pl.kernel,
    out_type=jax.ShapeDtypeStruct((NC * NPA, D), jnp.float32),
    mesh=_mesh,
    scratch_types=[
        pltpu.VMEM((2, BE), jnp.int32),
        pltpu.VMEM((2, BE), jnp.int32),
        pltpu.VMEM((BCH, KA), jnp.int32),
        pltpu.VMEM((BCH, KA), jnp.int32),
        pltpu.VMEM((KA, D), jnp.float32),
        pltpu.VMEM((KA, D), jnp.float32),
        pltpu.VMEM((KA, D), jnp.float32),
        pltpu.VMEM((KA, D), jnp.float32),
        pltpu.VMEM((KA, D), jnp.float32),
        pltpu.VMEM_SHARED((NPA, D), jnp.float32),
        pltpu.SemaphoreType.DMA,
        pltpu.SemaphoreType.DMA,
        pltpu.SemaphoreType.DMA,
        pltpu.SemaphoreType.DMA,
        pltpu.SemaphoreType.DMA,
        pltpu.SemaphoreType.DMA,
        pltpu.SemaphoreType.DMA,
        pltpu.SemaphoreType.DMA,
        pltpu.SemaphoreType.DMA,
        pltpu.SemaphoreType.DMA,
        pltpu.SemaphoreType.DMA,
    ],
)
def _agg_pass(g_hbm, ei_hbm, out_hbm, stage_a, stage_b, srcb, dstb,
              rows0, rows1, rows2, rows3, rows4, acc_sh, gs0, gs1, gs2, gs3,
              gs4, ss0, ss1, ss2, ss3, ss4, isem):
    c = lax.axis_index("c")
    s = lax.axis_index("s")
    wid = c * NS + s
    rows = (rows0, rows1, rows2, rows3, rows4)
    gs = (gs0, gs1, gs2, gs3, gs4)
    ss = (ss0, ss1, ss2, ss3, ss4)

    def zbody(i, carry):
        for jj in range(D // 16):
            rows0[i, pl.ds(jj * 16, 16)] = jnp.zeros((16,), jnp.float32)
            rows1[i, pl.ds(jj * 16, 16)] = jnp.zeros((16,), jnp.float32)
        return carry

    lax.fori_loop(0, KA, zbody, 0)
    for t in range(RPW // KA // 2):
        pltpu.sync_copy(rows0, acc_sh.at[pl.ds(s * RPW + (2 * t) * KA, KA)])
        pltpu.sync_copy(rows1,
                        acc_sh.at[pl.ds(s * RPW + (2 * t + 1) * KA, KA)])
    plsc.subcore_barrier()

    def _g(b, buf):
        return pltpu.async_copy(g_hbm.at[srcb.at[b]], rows[buf], gs[buf])

    def _wg(b, buf):
        pltpu.make_async_copy(g_hbm.at[srcb.at[b]], rows[buf],
                              gs[buf]).wait()

    def _s(b, buf):
        return pltpu.async_copy(rows[buf], acc_sh.at[dstb.at[b]], ss[buf],
                                add=True)

    def _ws(b, buf):
        pltpu.make_async_copy(rows[buf], acc_sh.at[dstb.at[b]],
                              ss[buf]).wait()

    def _repack1(ch, stg):
        for q in range(KA // 16):
            o = KA * ch + 16 * q
            srcb[ch, pl.ds(16 * q, 16)] = stg[0, pl.ds(o, 16)]
            dstb[ch, pl.ds(16 * q, 16)] = stg[1, pl.ds(o, 16)]

    def _pipeline(nch, stg):
        for p in range(NBUF - 1):
            _repack1(p, stg)
            _g(p, p)
        for b in range(nch):
            _wg(b, b % NBUF)
            _s(b, b % NBUF)
            nb = b + NBUF - 1
            if nb < nch:
                if b >= 1:
                    _ws(b - 1, (b - 1) % NBUF)
                _repack1(nb, stg)
                _g(nb, nb % NBUF)
        for b in range(max(0, nch - NBUF), nch):
            _ws(b, b % NBUF)

    hi = wid < NHI
    e0 = jnp.where(hi, wid * (RHI * K), NHI * RHI * K + (wid - NHI) * (RLO * K))

    def _fire_idx(base, stg):
        pltpu.async_copy(ei_hbm.at[pl.ds(0, 2), pl.ds(base, BE)], stg,
                         isem)

    def _wait_idx(stg):
        pltpu.make_async_copy(ei_hbm.at[pl.ds(0, 2), pl.ds(0, BE)], stg,
                              isem).wait()

    _fire_idx(e0, stage_a)

    def block2(i, carry):
        b0 = e0 + (2 * i) * BE
        _wait_idx(stage_a)
        _fire_idx(b0 + BE, stage_b)
        _pipeline(BCH, stage_a)
        _wait_idx(stage_b)
        _fire_idx(b0 + 2 * BE, stage_a)
        _pipeline(BCH, stage_b)
        return carry

    lax.fori_loop(0, (NB - 1) // 2, block2, 0)

    _wait_idx(stage_a)
    tbase = e0 + NB * BE
    pltpu.async_copy(ei_hbm.at[pl.ds(0, 2), pl.ds(tbase, (RLO - 72) * K)],
                     stage_b.at[pl.ds(0, 2), pl.ds(0, (RLO - 72) * K)], isem)
    _pipeline(BCH, stage_a)
    pltpu.make_async_copy(ei_hbm.at[pl.ds(0, 2), pl.ds(0, (RLO - 72) * K)],
                          stage_b.at[pl.ds(0, 2), pl.ds(0, (RLO - 72) * K)],
                          isem).wait()

    @pl.when(hi)
    def _tail_hi():
        pltpu.sync_copy(ei_hbm.at[pl.ds(0, 2), pl.ds(tbase + (RLO - 72) * K, K)],
                        stage_b.at[pl.ds(0, 2), pl.ds((RLO - 72) * K, K)])
        _pipeline((RHI - 72) * 2, stage_b)

    @pl.when(jnp.logical_not(hi))
    def _tail_lo():
        _pipeline((RLO - 72) * 2, stage_b)
    plsc.subcore_barrier()
    pltpu.sync_copy(acc_sh.at[pl.ds(s * RPW, RPW)],
                    out_hbm.at[pl.ds(c * NPA + s * RPW, RPW)])


def _dense0_body(x_ref, w_ref, h_ref):
    h_ref[...] = lax.dot_general(x_ref[...], w_ref[...],
                                 (((1,), (1,)), ((), ())),
                                 precision=lax.Precision.HIGHEST,
                                 preferred_element_type=jnp.float32)


_dense0 = pl.pallas_call(
    _dense0_body,
    out_shape=jax.ShapeDtypeStruct((N, D), jnp.float32),
)


def _dis_col(d_ref):
    deg = d_ref[0:1, 0:N] + d_ref[1:2, 0:N] + 1.0
    return lax.transpose(lax.rsqrt(deg), (1, 0))


def _dense1_body(h_ref, d_ref, g_ref):
    g_ref[...] = _dis_col(d_ref) * h_ref[...]


_dense1 = pl.pallas_call(
    _dense1_body,
    out_shape=jax.ShapeDtypeStruct((N, D), jnp.float32),
)


def _dense2_body(acc_ref, g_ref, d_ref, b_ref, o_ref):
    tot = acc_ref[0:N, :] + acc_ref[NPA:NPA + N, :] + g_ref[...]
    o_ref[...] = jnp.maximum(_dis_col(d_ref) * tot + b_ref[...], 0.0)


_dense2 = pl.pallas_call(
    _dense2_body,
    out_shape=jax.ShapeDtypeStruct((N, D), jnp.float32),
)


def kernel(x, edge_index, W, b):
    ei = edge_index.astype(jnp.int32)

    h = _dense0(x, W)
    degf = _deg_pass(ei)
    d2 = degf.reshape(NC, NP)

    g = _dense1(h, d2)
    accf = _agg_pass(g, ei)
    out = _dense2(accf, g, d2, b.reshape(1, D))
    return out
